# baseline (speedup 1.0000x reference)
import jax
import jax.numpy as jnp
from jax.experimental import pallas as pl
from jax.experimental.pallas import tpu as pltpu

N = 10000
NFEAT = 128
NHID = 16
NCLASS = 8

BM = 200

_SLOPE = (1.0 / 8.0 + 1.0 / 3.0) / 2.0


def _fused_kernel(adj_ref, x_ref, w1_ref, w2_ref, w3_ref, w4_ref,
                  b1_ref, b2_ref, b3_ref, b4_ref,
                  out_ref, s1_s, s2_s):
    p = pl.program_id(0)
    i = pl.program_id(1)

    @pl.when(jnp.logical_and(p == 0, i == 0))
    def _():
        s1_s[...] = jnp.dot(x_ref[...], w1_ref[...],
                            preferred_element_type=jnp.float32)

    a = adj_ref[...]
    xblk = x_ref[pl.ds(i * BM, BM), :]

    @pl.when(p == 0)
    def _():
        h = jnp.dot(a, s1_s[...], preferred_element_type=jnp.float32)
        h = h + b1_ref[...] + b2_ref[...]
        h = h + jnp.dot(xblk, w2_ref[...], preferred_element_type=jnp.float32)
        h = jnp.where(h >= 0, h, _SLOPE * h)
        s2_s[pl.ds(i * BM, BM), :] = jnp.dot(
            h, w3_ref[...], preferred_element_type=jnp.float32)

    @pl.when(p == 1)
    def _():
        o = jnp.dot(a, s2_s[...], preferred_element_type=jnp.float32)
        o = o + b3_ref[...] + b4_ref[...]
        o = o + jnp.dot(xblk, w4_ref[...], preferred_element_type=jnp.float32)
        out_ref[...] = o


def kernel(x, adj, W1, b1, W2, b2, W3, b3, W4, b4):
    b1r = b1.reshape(1, NHID)
    b2r = b2.reshape(1, NHID)
    b3r = b3.reshape(1, NCLASS)
    b4r = b4.reshape(1, NCLASS)

    nb = N // BM
    const = lambda p, i: (0, 0)

    out = pl.pallas_call(
        _fused_kernel,
        grid=(2, nb),
        in_specs=[
            pl.BlockSpec((BM, N), lambda p, i: (i, 0)),
            pl.BlockSpec((N, NFEAT), const),
            pl.BlockSpec((NFEAT, NHID), const),
            pl.BlockSpec((NFEAT, NHID), const),
            pl.BlockSpec((NHID, NCLASS), const),
            pl.BlockSpec((NFEAT, NCLASS), const),
            pl.BlockSpec((1, NHID), const),
            pl.BlockSpec((1, NHID), const),
            pl.BlockSpec((1, NCLASS), const),
            pl.BlockSpec((1, NCLASS), const),
        ],
        out_specs=pl.BlockSpec((BM, NCLASS), lambda p, i: (i * p, 0)),
        out_shape=jax.ShapeDtypeStruct((N, NCLASS), jnp.float32),
        scratch_shapes=[
            pltpu.VMEM((N, NHID), jnp.float32),
            pltpu.VMEM((N, NCLASS), jnp.float32),
        ],
    )(adj, x, W1, W2, W3, W4, b1r, b2r, b3r, b4r)

    return (out, W1, W2, W3, W4)


# speedup vs baseline: 1.0353x; 1.0353x over previous
import jax
import jax.numpy as jnp
from jax.experimental import pallas as pl

N = 10000
NFEAT = 128
NHID = 16
NCLASS = 8

BM = 200

_SLOPE = (1.0 / 8.0 + 1.0 / 3.0) / 2.0


def _small_mm_kernel(x_ref, w1_ref, w2_ref, w4_ref, b2_ref, b4_ref,
                     s1_ref, skip0_ref, skip1_ref):
    x = x_ref[...]
    s1_ref[...] = jnp.dot(x, w1_ref[...], preferred_element_type=jnp.float32)
    skip0_ref[...] = (
        jnp.dot(x, w2_ref[...], preferred_element_type=jnp.float32)
        + b2_ref[...])
    skip1_ref[...] = (
        jnp.dot(x, w4_ref[...], preferred_element_type=jnp.float32)
        + b4_ref[...])


def _pass_a_kernel(a_ref, s1_ref, skip0_ref, b1_ref, w3_ref, s2_ref):
    h = jnp.dot(a_ref[...], s1_ref[...], preferred_element_type=jnp.float32)
    h = h + b1_ref[...] + skip0_ref[...]
    h = jnp.where(h >= 0, h, _SLOPE * h)
    s2_ref[...] = jnp.dot(h, w3_ref[...], preferred_element_type=jnp.float32)


def _pass_b_kernel(a_ref, s2_ref, skip1_ref, b3_ref, out_ref):
    o = jnp.dot(a_ref[...], s2_ref[...], preferred_element_type=jnp.float32)
    out_ref[...] = o + b3_ref[...] + skip1_ref[...]


def kernel(x, adj, W1, b1, W2, b2, W3, b3, W4, b4):
    b1r = b1.reshape(1, NHID)
    b2r = b2.reshape(1, NHID)
    b3r = b3.reshape(1, NCLASS)
    b4r = b4.reshape(1, NCLASS)

    s1, skip0, skip1 = pl.pallas_call(
        _small_mm_kernel,
        out_shape=(
            jax.ShapeDtypeStruct((N, NHID), jnp.float32),
            jax.ShapeDtypeStruct((N, NHID), jnp.float32),
            jax.ShapeDtypeStruct((N, NCLASS), jnp.float32),
        ),
    )(x, W1, W2, W4, b2r, b4r)

    grid = (N // BM,)

    s2 = pl.pallas_call(
        _pass_a_kernel,
        grid=grid,
        in_specs=[
            pl.BlockSpec((BM, N), lambda i: (i, 0)),
            pl.BlockSpec((N, NHID), lambda i: (0, 0)),
            pl.BlockSpec((BM, NHID), lambda i: (i, 0)),
            pl.BlockSpec((1, NHID), lambda i: (0, 0)),
            pl.BlockSpec((NHID, NCLASS), lambda i: (0, 0)),
        ],
        out_specs=pl.BlockSpec((BM, NCLASS), lambda i: (i, 0)),
        out_shape=jax.ShapeDtypeStruct((N, NCLASS), jnp.float32),
    )(adj, s1, skip0, b1r, W3)

    out = pl.pallas_call(
        _pass_b_kernel,
        grid=grid,
        in_specs=[
            pl.BlockSpec((BM, N), lambda i: (i, 0)),
            pl.BlockSpec((N, NCLASS), lambda i: (0, 0)),
            pl.BlockSpec((BM, NCLASS), lambda i: (i, 0)),
            pl.BlockSpec((1, NCLASS), lambda i: (0, 0)),
        ],
        out_specs=pl.BlockSpec((BM, NCLASS), lambda i: (i, 0)),
        out_shape=jax.ShapeDtypeStruct((N, NCLASS), jnp.float32),
    )(adj, s2, skip1, b3r)

    return (out, W1, W2, W3, W4)
